# Initial kernel scaffold; baseline (speedup 1.0000x reference)
import jax
import jax.numpy as jnp
from jax.experimental import pallas as pl


def kernel(words, table):
    raise NotImplementedError("write your pallas kernel here")


# speedup vs baseline: 3.0036x; 3.0036x over previous
import functools

import jax
import jax.numpy as jnp
from jax import lax
from jax.experimental import pallas as pl
from jax.experimental.pallas import tpu as pltpu
from jax.experimental.pallas import tpu_sc as plsc

EMB = 300
BATCH = 16384
NUM_CORES = 2
NUM_SUBCORES = 16
NUM_WORKERS = NUM_CORES * NUM_SUBCORES
CHUNK = 128
PER_WORKER = BATCH // NUM_WORKERS
NUM_CHUNKS = PER_WORKER // CHUNK


def _build():
    mesh = plsc.VectorSubcoreMesh(core_axis_name="c", subcore_axis_name="s")

    @functools.partial(
        pl.kernel,
        mesh=mesh,
        out_type=jax.ShapeDtypeStruct((BATCH, EMB), jnp.float32),
        scratch_types=[
            pltpu.VMEM((PER_WORKER,), jnp.int32),
            pltpu.SMEM((PER_WORKER,), jnp.int32),
            pltpu.VMEM((2, CHUNK, EMB), jnp.float32),
            pltpu.SemaphoreType.DMA,
            pltpu.SemaphoreType.DMA,
            pltpu.SemaphoreType.DMA,
        ],
    )
    def emb_kernel(words_hbm, table_hbm, out_hbm, idx_v, idx_s, rows_v,
                   gsem0, gsem1, osem):
        wid = lax.axis_index("s") * NUM_CORES + lax.axis_index("c")
        base = wid * PER_WORKER
        del idx_s
        pltpu.sync_copy(words_hbm.at[pl.ds(base, PER_WORKER)], idx_v)

        gsems = (gsem0, gsem1)

        def fire(c):
            buf = c % 2
            for g in range(CHUNK // 16):
                vec = idx_v[pl.ds(c * CHUNK + g * 16, 16)]
                for l in range(16):
                    pltpu.async_copy(table_hbm.at[vec[l]],
                                     rows_v.at[buf, g * 16 + l], gsems[buf])

        def drain(c):
            buf = c % 2
            pltpu.make_async_copy(
                table_hbm.at[pl.ds(0, CHUNK)], rows_v.at[buf],
                gsems[buf]).wait()

        def write(c, blocking):
            buf = c % 2
            copy = pltpu.async_copy(
                rows_v.at[buf], out_hbm.at[pl.ds(base + c * CHUNK, CHUNK)],
                osem)
            if blocking:
                copy.wait()
            return copy

        fire(0)
        pending = None
        for c in range(1, NUM_CHUNKS):
            drain(c - 1)
            if pending is not None:
                pending.wait()
            fire(c)
            pending = write(c - 1, blocking=False)
        drain(NUM_CHUNKS - 1)
        if pending is not None:
            pending.wait()
        write(NUM_CHUNKS - 1, blocking=True)

    return emb_kernel


_emb_lookup = _build()


def kernel(words, table):
    return _emb_lookup(words.astype(jnp.int32), table)


# speedup vs baseline: 3.0312x; 1.0092x over previous
import functools

import jax
import jax.numpy as jnp
from jax import lax
from jax.experimental import pallas as pl
from jax.experimental.pallas import tpu as pltpu
from jax.experimental.pallas import tpu_sc as plsc

EMB = 300
BATCH = 16384
NUM_CORES = 2
NUM_SUBCORES = 16
NUM_WORKERS = NUM_CORES * NUM_SUBCORES
CHUNK = 128
PER_WORKER = BATCH // NUM_WORKERS
NUM_CHUNKS = PER_WORKER // CHUNK


def _build():
    mesh = plsc.VectorSubcoreMesh(core_axis_name="c", subcore_axis_name="s")

    @functools.partial(
        pl.kernel,
        mesh=mesh,
        out_type=jax.ShapeDtypeStruct((BATCH, EMB), jnp.float32),
        scratch_types=[
            pltpu.VMEM((PER_WORKER,), jnp.int32),
            pltpu.SMEM((PER_WORKER,), jnp.int32),
            pltpu.VMEM((2, CHUNK, EMB), jnp.float32),
            pltpu.SemaphoreType.DMA,
            pltpu.SemaphoreType.DMA,
            pltpu.SemaphoreType.DMA,
        ],
    )
    def emb_kernel(words_hbm, table_hbm, out_hbm, idx_v, idx_s, rows_v,
                   gsem0, gsem1, osem):
        wid = lax.axis_index("s") * NUM_CORES + lax.axis_index("c")
        base = wid * PER_WORKER
        del idx_s
        pltpu.sync_copy(words_hbm.at[pl.ds(base, PER_WORKER)], idx_v)

        gsems = (gsem0, gsem1)

        def fire(c):
            buf = c % 2
            for g in range(CHUNK // 16):
                vec = idx_v[pl.ds(c * CHUNK + g * 16, 16)]
                for l in range(16):
                    pltpu.async_copy(table_hbm.at[vec[l]],
                                     rows_v.at[buf, g * 16 + l], gsems[buf])

        def drain(c):
            buf = c % 2
            pltpu.make_async_copy(
                table_hbm.at[pl.ds(0, CHUNK)], rows_v.at[buf],
                gsems[buf]).wait()

        def write(c, blocking):
            buf = c % 2
            copy = pltpu.async_copy(
                rows_v.at[buf], out_hbm.at[pl.ds(base + c * CHUNK, CHUNK)],
                osem)
            if blocking:
                copy.wait()
            return copy

        fire(0)
        fire(1)
        for c in range(NUM_CHUNKS):
            drain(c)
            write(c, blocking=True)
            if c + 2 < NUM_CHUNKS:
                fire(c + 2)

    return emb_kernel


_emb_lookup = _build()


def kernel(words, table):
    return _emb_lookup(words.astype(jnp.int32), table)
